# baseline (speedup 1.0000x reference)
import jax
import jax.numpy as jnp
from jax.experimental import pallas as pl
from jax.experimental.pallas import tpu as pltpu

_NUM_PROPOSALS = 300
_SCORE_THR = 0.85
_NMS_THR = 0.1
_NEG = float("-inf")

_C_PAD = 1024
_B_PAD = 20480
_TOT = _C_PAD + _B_PAD
_ROWS = _TOT // 128
_N_CHUNKS = 10


def _row_gather_body(oid_ref, cache_ref, out_ref, sem):
    cp = pltpu.make_async_copy(cache_ref.at[oid_ref[0]], out_ref, sem)
    cp.start()
    cp.wait()


def _nms_update_body(oid_ref, x_ref, cache_ref, out_ref, newcache_ref,
                     rowbuf, copy_sem, row_sem):
    n_img = cache_ref.shape[0]
    chunk = n_img // _N_CHUNKS
    copies = [
        pltpu.make_async_copy(cache_ref.at[pl.ds(k * chunk, chunk)],
                              newcache_ref.at[pl.ds(k * chunk, chunk)],
                              copy_sem)
        for k in range(_N_CHUNKS)
    ]
    for cp in copies:
        cp.start()

    x1 = x_ref[0]
    y1 = x_ref[1]
    x2 = x_ref[2]
    y2 = x_ref[3]
    s = x_ref[4]
    areas = (x2 - x1) * (y2 - y1)
    rows = jax.lax.broadcasted_iota(jnp.int32, (_ROWS, 128), 0)
    lanes = jax.lax.broadcasted_iota(jnp.int32, (_ROWS, 128), 1)
    flat = rows * 128 + lanes
    big = jnp.int32(2**30)

    rowbuf[...] = jnp.zeros_like(rowbuf)

    act = jnp.where(s > _SCORE_THR, s, _NEG)
    have = jnp.max(act) > _NEG
    gmax = jnp.max(s)
    fb = jnp.min(jnp.where(s == gmax, flat, big))
    act = jnp.where(have, act, jnp.where(flat == fb, s, _NEG))

    def cond(carry):
        _, m, _, _ = carry
        return m > _NEG

    def body(carry):
        act, m, keep, cnt = carry
        pick = jnp.min(jnp.where(act == m, flat, big))
        onehot = flat == pick
        px1 = jnp.max(jnp.where(onehot, x1, _NEG))
        py1 = jnp.max(jnp.where(onehot, y1, _NEG))
        px2 = jnp.max(jnp.where(onehot, x2, _NEG))
        py2 = jnp.max(jnp.where(onehot, y2, _NEG))
        pa = jnp.max(jnp.where(onehot, areas, _NEG))
        xx1 = jnp.maximum(px1, x1)
        yy1 = jnp.maximum(py1, y1)
        xx2 = jnp.minimum(px2, x2)
        yy2 = jnp.minimum(py2, y2)
        inter = jnp.maximum(xx2 - xx1, 0.0) * jnp.maximum(yy2 - yy1, 0.0)
        iou = inter / (pa + areas - inter + 1e-12)
        nact = jnp.where((iou > _NMS_THR) | onehot, _NEG, act)
        nkeep = jnp.where(onehot, 1.0, keep)

        @pl.when(cnt < _NUM_PROPOSALS)
        def _():
            for c, v in enumerate((px1, py1, px2, py2, m)):
                rowbuf[pl.ds(cnt, 1), c:c + 1] = jnp.full((1, 1), v,
                                                          jnp.float32)

        return nact, jnp.max(nact), nkeep, cnt + jnp.int32(1)

    init = (act, jnp.max(act), jnp.zeros((_ROWS, 128), jnp.float32),
            jnp.int32(0))
    _, _, keepf, _ = jax.lax.while_loop(cond, body, init)
    keep = keepf > 0.0
    out_ref[0] = jnp.where(keep, x1, 0.0)
    out_ref[1] = jnp.where(keep, y1, 0.0)
    out_ref[2] = jnp.where(keep, x2, 0.0)
    out_ref[3] = jnp.where(keep, y2, 0.0)
    out_ref[4] = jnp.where(keep, s, 0.0)

    for cp in copies:
        cp.wait()
    rcp = pltpu.make_async_copy(rowbuf, newcache_ref.at[oid_ref[0]], row_sem)
    rcp.start()
    rcp.wait()


def _plane(cvals, bvals, fill):
    return jnp.concatenate([
        cvals,
        jnp.full((_C_PAD - _NUM_PROPOSALS,), fill, jnp.float32),
        bvals,
        jnp.full((_B_PAD - bvals.shape[0],), fill, jnp.float32),
    ])


def kernel(bboxes, scores, boxes_cache, ordered_id):
    n_img = boxes_cache.shape[0]
    n_box = bboxes.shape[0]
    oid = jnp.asarray(ordered_id, jnp.int32).reshape((1,))

    row2 = pl.pallas_call(
        _row_gather_body,
        in_specs=[
            pl.BlockSpec(memory_space=pltpu.MemorySpace.SMEM),
            pl.BlockSpec(memory_space=pl.ANY),
        ],
        out_shape=jax.ShapeDtypeStruct((_NUM_PROPOSALS, 5), jnp.float32),
        scratch_shapes=[pltpu.SemaphoreType.DMA],
    )(oid, boxes_cache)

    x = jnp.stack([
        _plane(row2[:, 0], bboxes[:, 0], 0.0),
        _plane(row2[:, 1], bboxes[:, 1], 0.0),
        _plane(row2[:, 2], bboxes[:, 2], 0.0),
        _plane(row2[:, 3], bboxes[:, 3], 0.0),
        _plane(row2[:, 4], scores, _NEG),
    ]).reshape(5, _ROWS, 128)

    outm, new_cache = pl.pallas_call(
        _nms_update_body,
        in_specs=[
            pl.BlockSpec(memory_space=pltpu.MemorySpace.SMEM),
            pl.BlockSpec(memory_space=pltpu.MemorySpace.VMEM),
            pl.BlockSpec(memory_space=pl.ANY),
        ],
        out_shape=(
            jax.ShapeDtypeStruct((5, _ROWS, 128), jnp.float32),
            jax.ShapeDtypeStruct(boxes_cache.shape, jnp.float32),
        ),
        out_specs=(
            pl.BlockSpec(memory_space=pltpu.MemorySpace.VMEM),
            pl.BlockSpec(memory_space=pl.ANY),
        ),
        scratch_shapes=[
            pltpu.VMEM((_NUM_PROPOSALS, 5), jnp.float32),
            pltpu.SemaphoreType.DMA,
            pltpu.SemaphoreType.DMA,
        ],
    )(oid, x, boxes_cache)

    flatm = outm.reshape(5, _TOT)
    merged = jnp.concatenate(
        [flatm[:, :_NUM_PROPOSALS], flatm[:, _C_PAD:_C_PAD + n_box]], axis=1)
    out_boxes = merged[:4].T
    out_scores = merged[4]

    return out_boxes, out_scores, new_cache
